# Initial kernel scaffold; baseline (speedup 1.0000x reference)
import jax
import jax.numpy as jnp
from jax.experimental import pallas as pl


def kernel(logits, prev_scores):
    raise NotImplementedError("write your pallas kernel here")


# speedup vs baseline: 22.6283x; 22.6283x over previous
import functools

import jax
import jax.numpy as jnp
from jax import lax
from jax.experimental import pallas as pl

_TOPK = 8
_NUM_DRAFT = 48
_NEG_INF = float("-inf")
_BIG_I32 = 2**30


def _stage1_body(S, L, x_ref, scores_ref, tokens_ref):
    x = x_ref[0]
    sm = jnp.max(x, axis=1, keepdims=True)
    m0 = jnp.max(sm)
    lse = jnp.log(jnp.sum(jnp.exp(x - m0)))

    strip_iota = lax.broadcasted_iota(jnp.int32, (S, 1), 0)
    lane_iota = lax.broadcasted_iota(jnp.int32, (1, L), 1)
    out_iota = lax.broadcasted_iota(jnp.int32, (1, _TOPK), 1)

    vals = jnp.zeros((1, _TOPK), jnp.float32)
    toks = jnp.zeros((1, _TOPK), jnp.int32)
    for i in range(_TOPK):
        m = jnp.max(sm)
        s_star = jnp.min(jnp.where(sm == m, strip_iota, _BIG_I32))
        row = x_ref[0, pl.ds(s_star, 1), :]
        j_star = jnp.min(jnp.where(row == m, lane_iota, _BIG_I32))
        vals = jnp.where(out_iota == i, m, vals)
        toks = jnp.where(out_iota == i, s_star * L + j_star, toks)
        new_row = jnp.where(lane_iota == j_star, _NEG_INF, row)
        x_ref[0, pl.ds(s_star, 1), :] = new_row
        sm = jnp.where(strip_iota == s_star, jnp.max(new_row), sm)

    scores_ref[0] = (vals - m0) - lse
    tokens_ref[0] = toks


def _stage2_body(KK, scores_ref, tokens_ref, prev_ref, out_s_ref, out_t_ref):
    cum = scores_ref[...] + prev_ref[...]
    toks = tokens_ref[...]
    B = cum.shape[0]
    lane_kk = lax.broadcasted_iota(jnp.int32, (B, KK), 1)
    lane_t = lax.broadcasted_iota(jnp.int32, (B, _NUM_DRAFT), 1)

    def body(i, carry):
        cum, outv, outt = carry
        m = jnp.max(cum, axis=1, keepdims=True)
        jsel = jnp.min(jnp.where(cum == m, lane_kk, _BIG_I32), axis=1,
                       keepdims=True)
        tok = jnp.min(jnp.where(lane_kk == jsel, toks, _BIG_I32), axis=1,
                      keepdims=True)
        outv = jnp.where(lane_t == i, m, outv)
        outt = jnp.where(lane_t == i, tok, outt)
        cum = jnp.where(lane_kk == jsel, _NEG_INF, cum)
        return cum, outv, outt

    outv = jnp.zeros((B, _NUM_DRAFT), jnp.float32)
    outt = jnp.zeros((B, _NUM_DRAFT), jnp.int32)
    _, outv, outt = lax.fori_loop(0, _NUM_DRAFT, body, (cum, outv, outt))
    out_s_ref[...] = outv
    out_t_ref[...] = outt


def kernel(logits, prev_scores):
    B, K, V = logits.shape
    if V % 1000 == 0:
        L = 1000
    else:
        L = V
    S = V // L
    R = B * K

    x3 = logits.reshape(R, S, L)
    scores, tokens = pl.pallas_call(
        functools.partial(_stage1_body, S, L),
        grid=(R,),
        in_specs=[pl.BlockSpec((1, S, L), lambda i: (i, 0, 0))],
        out_specs=[
            pl.BlockSpec((1, 1, K), lambda i: (i, 0, 0)),
            pl.BlockSpec((1, 1, K), lambda i: (i, 0, 0)),
        ],
        out_shape=[
            jax.ShapeDtypeStruct((R, 1, K), jnp.float32),
            jax.ShapeDtypeStruct((R, 1, K), jnp.int32),
        ],
    )(x3)

    s2 = scores.reshape(B, K * K)
    t2 = tokens.reshape(B, K * K)
    prev_rep = jnp.repeat(prev_scores, K, axis=1)

    top_s, top_t = pl.pallas_call(
        functools.partial(_stage2_body, K * K),
        out_shape=[
            jax.ShapeDtypeStruct((B, _NUM_DRAFT), jnp.float32),
            jax.ShapeDtypeStruct((B, _NUM_DRAFT), jnp.int32),
        ],
    )(s2, t2, prev_rep)
    return top_s, top_t
